# baseline (speedup 1.0000x reference)
import functools

import jax
import jax.numpy as jnp
from jax import lax
from jax.experimental import pallas as pl
from jax.experimental.pallas import tpu as pltpu
from jax.experimental.pallas import tpu_sc as plsc

N = 10000
E = 320000
F_IN = 128
H1 = 8
C1 = 16
HID = H1 * C1
NC2 = 40
R1 = HID + 16
R2 = 48
NP = 10240
NWORK = 32
EPW = E // NWORK
CHUNK = 80
NCH = EPW // CHUNK
RPT = NP // 16
ZR = 128


def _mm1_body(x_ref, wa_ref, wd_ref, t1_ref, om_ref):
    xb = x_ref[...]
    t1_ref[...] = jnp.dot(xb, wa_ref[...], preferred_element_type=jnp.float32)
    om_ref[...] = jnp.dot(xb, wd_ref[...], preferred_element_type=jnp.float32)


def _merge1_body(p_ref, e8_ref, w2a_ref, w2d_ref, b1_ref, t2_ref, om2_ref):
    p = p_ref[...]
    s = p[0] + p[1]
    num = s[:, :HID]
    den = s[:, HID:HID + H1]
    dene = jnp.dot(den, e8_ref[...], preferred_element_type=jnp.float32)
    x2 = num / (dene + 1e-16) + b1_ref[...]
    x2 = jnp.where(x2 > 0.0, x2, jnp.exp(x2) - 1.0)
    t2_ref[...] = jnp.dot(x2, w2a_ref[...], preferred_element_type=jnp.float32)
    om2_ref[...] = jnp.dot(x2, w2d_ref[...], preferred_element_type=jnp.float32)


def _final_body(p_ref, b2_ref, o_ref):
    p = p_ref[...]
    s = p[0] + p[1]
    num = s[:, :NC2]
    den = s[:, NC2:NC2 + 1]
    o = num / (den + 1e-16) + b2_ref[...]
    m = jnp.max(o, axis=1, keepdims=True)
    z = o - m
    o_ref[...] = z - jnp.log(jnp.sum(jnp.exp(z), axis=1, keepdims=True))


def _edge_body_l1(ts, od, lane, n):
    @plsc.parallel_loop(0, n, unroll=8)
    def _(e):
        tv = ts[e, pl.ds(HID, 16)] + od[e, :]
        tv = jnp.where(tv > 0.0, tv, 0.2 * tv)
        w16 = jnp.exp(tv)
        ts[e, pl.ds(HID, 16)] = w16
        for k in range(H1):
            ts[e, pl.ds(k * 16, 16)] = ts[e, pl.ds(k * 16, 16)] * w16[k]


def _edge_body_l2(ts, od, lane, n):
    @plsc.parallel_loop(0, n, unroll=8)
    def _(e):
        tv = ts[e, pl.ds(32, 16)] + od[e, :]
        tv = jnp.where(tv > 0.0, tv, 0.2 * tv)
        w16 = jnp.exp(tv)
        w = w16[8]
        ts[e, pl.ds(0, 16)] = ts[e, pl.ds(0, 16)] * w
        ts[e, pl.ds(16, 16)] = ts[e, pl.ds(16, 16)] * w
        v2 = ts[e, pl.ds(32, 16)] * w
        ts[e, pl.ds(32, 16)] = jnp.where(lane == 8, w16, v2)


def _make_edge_kernel(row, body, chunk):
    nfull = EPW // chunk
    tail = EPW - nfull * chunk

    def kern(tab_hbm, om_hbm, src_hbm, dst_hbm, out_hbm,
             sidx0, didx0, ts0, od0, sd0, sidx1, didx1, ts1, od1, sd1,
             tsi, tdi, acc, semi0, semg0, sems0, semi1, semg1, sems1):
        cid = lax.axis_index("c")
        sid = lax.axis_index("s")
        wid = cid * 16 + sid
        lane = lax.broadcasted_iota(jnp.int32, (16,), 0)

        def zrow(r, _):
            for v in range(row // 16):
                ts0[r, pl.ds(v * 16, 16)] = jnp.zeros((16,), jnp.float32)
            return 0
        lax.fori_loop(0, 80, zrow, 0)
        for j in range(RPT // 80):
            pltpu.sync_copy(ts0.at[pl.ds(0, 80)],
                            acc.at[pl.ds(sid * RPT + j * 80, 80)])
        plsc.subcore_barrier()

        base0 = wid * EPW
        BUF = ((sidx0, didx0, ts0, od0, sd0, semi0, semg0, sems0),
               (sidx1, didx1, ts1, od1, sd1, semi1, semg1, sems1))

        def snap_didx(p):
            d_ = BUF[p][1]
            sd_ = BUF[p][4]
            for v in range(chunk // 16):
                sd_[pl.ds(v * 16, 16)] = d_[pl.ds(v * 16, 16)]

        def issue_idx(ci, p):
            b = base0 + ci * chunk
            s_, d_, _, _, _, si, _, _ = BUF[p]
            pltpu.async_copy(src_hbm.at[pl.ds(b, chunk)], s_, si)
            pltpu.async_copy(dst_hbm.at[pl.ds(b, chunk)], d_, si)

        def wait_idx(p):
            s_, d_, _, _, _, si, _, _ = BUF[p]
            pltpu.make_async_copy(src_hbm.at[pl.ds(0, chunk)], s_, si).wait()
            pltpu.make_async_copy(dst_hbm.at[pl.ds(0, chunk)], d_, si).wait()

        def issue_gathers(p):
            s_, d_, ts_, od_, _, _, sg, _ = BUF[p]
            pltpu.async_copy(tab_hbm.at[s_], ts_, sg)
            pltpu.async_copy(om_hbm.at[d_], od_, sg)

        def wait_gathers(p):
            s_, d_, ts_, od_, _, _, sg, _ = BUF[p]
            pltpu.make_async_copy(tab_hbm.at[s_], ts_, sg).wait()
            pltpu.make_async_copy(om_hbm.at[d_], od_, sg).wait()

        def issue_scatter(p):
            ts_ = BUF[p][2]
            sd_ = BUF[p][4]
            ss = BUF[p][7]
            pltpu.async_copy(ts_, acc.at[sd_], ss, add=True)

        def wait_scatter(p):
            ts_ = BUF[p][2]
            sd_ = BUF[p][4]
            ss = BUF[p][7]
            pltpu.make_async_copy(ts_, acc.at[sd_], ss).wait()

        issue_idx(0, 0)
        wait_idx(0)
        issue_gathers(0)
        if nfull > 1:
            issue_idx(1, 1)

        def step(ci, p):
            q = 1 - p
            wait_gathers(p)
            snap_didx(p)

            @pl.when(ci >= 1)
            def _():
                wait_scatter(q)

            @pl.when(ci + 1 < nfull)
            def _():
                wait_idx(q)
                issue_gathers(q)

            @pl.when(ci + 2 < nfull)
            def _():
                issue_idx(ci + 2, p)
            body(BUF[p][2], BUF[p][3], lane, chunk)
            issue_scatter(p)

        def loop_body(i, _):
            step(2 * i, 0)
            step(2 * i + 1, 1)
            return 0
        lax.fori_loop(0, nfull // 2, loop_body, 0)

        if nfull % 2:
            ci = nfull - 1
            wait_gathers(0)
            snap_didx(0)
            if ci >= 1:
                wait_scatter(1)
            body(ts0, od0, lane, chunk)
            issue_scatter(0)
            wait_scatter(0)
        else:
            wait_scatter(1)

        if tail:
            b = base0 + nfull * chunk
            pltpu.sync_copy(src_hbm.at[pl.ds(b, tail)], tsi)
            pltpu.sync_copy(dst_hbm.at[pl.ds(b, tail)], tdi)
            pltpu.async_copy(tab_hbm.at[tsi], ts0.at[pl.ds(0, tail)],
                             semg0).wait()
            pltpu.async_copy(om_hbm.at[tdi], od0.at[pl.ds(0, tail)],
                             semg0).wait()
            body(ts0, od0, lane, tail)
            pltpu.async_copy(ts0.at[pl.ds(0, tail)], acc.at[tdi], sems0,
                             add=True).wait()

        plsc.subcore_barrier()
        for j in range(RPT // ZR):
            r0 = sid * RPT + j * ZR
            pltpu.sync_copy(acc.at[pl.ds(r0, ZR)], out_hbm.at[cid, pl.ds(r0, ZR)])

    return kern


@functools.cache
def _build_edge_kernels():
    mesh = plsc.VectorSubcoreMesh(core_axis_name="c", subcore_axis_name="s")
    params = pltpu.CompilerParams(use_tc_tiling_on_sc=False)

    def make(row, body, chunk):
        tail = EPW - (EPW // chunk) * chunk
        scratch = [
            pltpu.VMEM((chunk,), jnp.int32),
            pltpu.VMEM((chunk,), jnp.int32),
            pltpu.VMEM((chunk, row), jnp.float32),
            pltpu.VMEM((chunk, 16), jnp.float32),
            pltpu.VMEM((chunk,), jnp.int32),
            pltpu.VMEM((chunk,), jnp.int32),
            pltpu.VMEM((chunk,), jnp.int32),
            pltpu.VMEM((chunk, row), jnp.float32),
            pltpu.VMEM((chunk, 16), jnp.float32),
            pltpu.VMEM((chunk,), jnp.int32),
            pltpu.VMEM((max(tail, 8),), jnp.int32),
            pltpu.VMEM((max(tail, 8),), jnp.int32),
            pltpu.VMEM_SHARED((NP, row), jnp.float32),
        ]
        scratch += [pltpu.SemaphoreType.DMA] * 6
        return functools.partial(
            pl.kernel,
            compiler_params=params,
            out_type=jax.ShapeDtypeStruct((2, NP, row), jnp.float32),
            mesh=mesh,
            scratch_types=scratch,
        )(_make_edge_kernel(row, body, chunk))

    edge1 = make(R1, _edge_body_l1, 96)
    edge2 = make(R2, _edge_body_l2, 128)
    return edge1, edge2


def kernel(x, edge_index, W1, att_src1, att_dst1, b1, W2, att_src2, att_dst2, b2):
    f32 = jnp.float32
    W1r = W1.reshape(F_IN, H1, C1)
    S1 = jnp.einsum("fkc,kc->fk", W1r, att_src1)
    D1 = jnp.einsum("fkc,kc->fk", W1r, att_dst1)
    W1aug = jnp.concatenate([W1, S1, D1], axis=1)
    W1d = jnp.concatenate([D1, jnp.zeros((F_IN, 8), f32)], axis=1)

    vs2 = W2 @ att_src2[0]
    vd2 = W2 @ att_dst2[0]
    W2aug = jnp.concatenate(
        [W2, vs2[:, None], jnp.zeros((HID, R2 - NC2 - 1), f32)], axis=1)
    W2d = jnp.concatenate(
        [jnp.zeros((HID, 8), f32), vd2[:, None], jnp.zeros((HID, 7), f32)],
        axis=1)

    E8 = jnp.repeat(jnp.eye(H1, dtype=f32), C1, axis=1)

    bn = 2000
    grid = N // bn

    t1, om1 = pl.pallas_call(
        _mm1_body,
        grid=(grid,),
        in_specs=[
            pl.BlockSpec((bn, F_IN), lambda i: (i, 0)),
            pl.BlockSpec((F_IN, R1), lambda i: (0, 0)),
            pl.BlockSpec((F_IN, 16), lambda i: (0, 0)),
        ],
        out_specs=[
            pl.BlockSpec((bn, R1), lambda i: (i, 0)),
            pl.BlockSpec((bn, 16), lambda i: (i, 0)),
        ],
        out_shape=[
            jax.ShapeDtypeStruct((N, R1), f32),
            jax.ShapeDtypeStruct((N, 16), f32),
        ],
    )(x, W1aug, W1d)

    edge1, edge2 = _build_edge_kernels()
    src, dst = edge_index[0], edge_index[1]
    parts1 = edge1(t1, om1, src, dst)

    t2, om2 = pl.pallas_call(
        _merge1_body,
        grid=(grid,),
        in_specs=[
            pl.BlockSpec((2, bn, R1), lambda i: (0, i, 0)),
            pl.BlockSpec((H1, HID), lambda i: (0, 0)),
            pl.BlockSpec((HID, R2), lambda i: (0, 0)),
            pl.BlockSpec((HID, 16), lambda i: (0, 0)),
            pl.BlockSpec((1, HID), lambda i: (0, 0)),
        ],
        out_specs=[
            pl.BlockSpec((bn, R2), lambda i: (i, 0)),
            pl.BlockSpec((bn, 16), lambda i: (i, 0)),
        ],
        out_shape=[
            jax.ShapeDtypeStruct((N, R2), f32),
            jax.ShapeDtypeStruct((N, 16), f32),
        ],
    )(parts1, E8, W2aug, W2d, b1[None, :])

    parts2 = edge2(t2, om2, src, dst)

    out = pl.pallas_call(
        _final_body,
        grid=(grid,),
        in_specs=[
            pl.BlockSpec((2, bn, R2), lambda i: (0, i, 0)),
            pl.BlockSpec((1, NC2), lambda i: (0, 0)),
        ],
        out_specs=pl.BlockSpec((bn, NC2), lambda i: (i, 0)),
        out_shape=jax.ShapeDtypeStruct((N, NC2), f32),
    )(parts2, b2[None, :])

    return out


# speedup vs baseline: 1.2104x; 1.2104x over previous
import functools

import jax
import jax.numpy as jnp
from jax import lax
from jax.experimental import pallas as pl
from jax.experimental.pallas import tpu as pltpu
from jax.experimental.pallas import tpu_sc as plsc

N = 10000
E = 320000
F_IN = 128
H1 = 8
C1 = 16
HID = H1 * C1
NC2 = 40
R1 = HID + 16
R2 = 48
NP = 10240
NWORK = 32
EPW = E // NWORK
CHUNK = 80
NCH = EPW // CHUNK
RPT = NP // 16
ZR = 128


def _mm1_body(x_ref, wa_ref, wd_ref, t1_ref, om_ref):
    xb = x_ref[...]
    t1_ref[...] = jnp.dot(xb, wa_ref[...], preferred_element_type=jnp.float32)
    om_ref[...] = jnp.dot(xb, wd_ref[...], preferred_element_type=jnp.float32)


def _merge1_body(p_ref, e8_ref, w2a_ref, w2d_ref, b1_ref, t2_ref, om2_ref):
    p = p_ref[...]
    s = p[0] + p[1]
    num = s[:, :HID]
    den = s[:, HID:HID + H1]
    dene = jnp.dot(den, e8_ref[...], preferred_element_type=jnp.float32)
    x2 = num / (dene + 1e-16) + b1_ref[...]
    x2 = jnp.where(x2 > 0.0, x2, jnp.exp(x2) - 1.0)
    t2_ref[...] = jnp.dot(x2, w2a_ref[...], preferred_element_type=jnp.float32)
    om2_ref[...] = jnp.dot(x2, w2d_ref[...], preferred_element_type=jnp.float32)


def _final_body(p_ref, b2_ref, o_ref):
    p = p_ref[...]
    s = p[0] + p[1]
    num = s[:, :NC2]
    den = s[:, NC2:NC2 + 1]
    o = num / (den + 1e-16) + b2_ref[...]
    m = jnp.max(o, axis=1, keepdims=True)
    z = o - m
    o_ref[...] = z - jnp.log(jnp.sum(jnp.exp(z), axis=1, keepdims=True))


_GDN = lax.GatherDimensionNumbers(
    offset_dims=(), collapsed_slice_dims=(0,), start_index_map=(0,))


def _lane_bcast(v, k):
    idx = jnp.full((16, 1), k, dtype=jnp.int32)
    return lax.gather(v, idx, _GDN, (1,),
                      mode=lax.GatherScatterMode.PROMISE_IN_BOUNDS)


def _edge_body_l1(ts, od, lane, n):
    @plsc.parallel_loop(0, n, unroll=4)
    def _(e):
        tv = ts[e, pl.ds(HID, 16)] + od[e, :]
        tv = jnp.where(tv > 0.0, tv, 0.2 * tv)
        w16 = jnp.exp(tv)
        ts[e, pl.ds(HID, 16)] = w16
        for k in range(H1):
            wk = _lane_bcast(w16, k)
            ts[e, pl.ds(k * 16, 16)] = ts[e, pl.ds(k * 16, 16)] * wk


def _edge_body_l2(ts, od, lane, n):
    @plsc.parallel_loop(0, n, unroll=4)
    def _(e):
        tv = ts[e, pl.ds(32, 16)] + od[e, :]
        tv = jnp.where(tv > 0.0, tv, 0.2 * tv)
        w16 = jnp.exp(tv)
        w = w16[8]
        ts[e, pl.ds(0, 16)] = ts[e, pl.ds(0, 16)] * w
        ts[e, pl.ds(16, 16)] = ts[e, pl.ds(16, 16)] * w
        v2 = ts[e, pl.ds(32, 16)] * w
        ts[e, pl.ds(32, 16)] = jnp.where(lane == 8, w16, v2)


def _make_edge_kernel(row, body, chunk):
    nfull = EPW // chunk
    tail = EPW - nfull * chunk

    def kern(tab_hbm, om_hbm, src_hbm, dst_hbm, out_hbm,
             sidx0, didx0, ts0, od0, sd0, sidx1, didx1, ts1, od1, sd1,
             tsi, tdi, acc, semi0, semg0, sems0, semi1, semg1, sems1):
        cid = lax.axis_index("c")
        sid = lax.axis_index("s")
        wid = cid * 16 + sid
        lane = lax.broadcasted_iota(jnp.int32, (16,), 0)

        def zrow(r, _):
            for v in range(row // 16):
                ts0[r, pl.ds(v * 16, 16)] = jnp.zeros((16,), jnp.float32)
            return 0
        lax.fori_loop(0, 80, zrow, 0)
        for j in range(RPT // 80):
            pltpu.sync_copy(ts0.at[pl.ds(0, 80)],
                            acc.at[pl.ds(sid * RPT + j * 80, 80)])
        plsc.subcore_barrier()

        base0 = wid * EPW
        BUF = ((sidx0, didx0, ts0, od0, sd0, semi0, semg0, sems0),
               (sidx1, didx1, ts1, od1, sd1, semi1, semg1, sems1))

        def snap_didx(p):
            d_ = BUF[p][1]
            sd_ = BUF[p][4]
            for v in range(chunk // 16):
                sd_[pl.ds(v * 16, 16)] = d_[pl.ds(v * 16, 16)]

        def issue_idx(ci, p):
            b = base0 + ci * chunk
            s_, d_, _, _, _, si, _, _ = BUF[p]
            pltpu.async_copy(src_hbm.at[pl.ds(b, chunk)], s_, si)
            pltpu.async_copy(dst_hbm.at[pl.ds(b, chunk)], d_, si)

        def wait_idx(p):
            s_, d_, _, _, _, si, _, _ = BUF[p]
            pltpu.make_async_copy(src_hbm.at[pl.ds(0, chunk)], s_, si).wait()
            pltpu.make_async_copy(dst_hbm.at[pl.ds(0, chunk)], d_, si).wait()

        def issue_gathers(p):
            s_, d_, ts_, od_, _, _, sg, _ = BUF[p]
            pltpu.async_copy(tab_hbm.at[s_], ts_, sg)
            pltpu.async_copy(om_hbm.at[d_], od_, sg)

        def wait_gathers(p):
            s_, d_, ts_, od_, _, _, sg, _ = BUF[p]
            pltpu.make_async_copy(tab_hbm.at[s_], ts_, sg).wait()
            pltpu.make_async_copy(om_hbm.at[d_], od_, sg).wait()

        def issue_scatter(p):
            ts_ = BUF[p][2]
            sd_ = BUF[p][4]
            ss = BUF[p][7]
            pltpu.async_copy(ts_, acc.at[sd_], ss, add=True)

        def wait_scatter(p):
            ts_ = BUF[p][2]
            sd_ = BUF[p][4]
            ss = BUF[p][7]
            pltpu.make_async_copy(ts_, acc.at[sd_], ss).wait()

        issue_idx(0, 0)
        wait_idx(0)
        issue_gathers(0)
        if nfull > 1:
            issue_idx(1, 1)

        def step(ci, p):
            q = 1 - p
            wait_gathers(p)
            snap_didx(p)

            @pl.when(ci >= 1)
            def _():
                wait_scatter(q)

            @pl.when(ci + 1 < nfull)
            def _():
                wait_idx(q)
                issue_gathers(q)

            @pl.when(ci + 2 < nfull)
            def _():
                issue_idx(ci + 2, p)
            body(BUF[p][2], BUF[p][3], lane, chunk)
            issue_scatter(p)

        def loop_body(i, _):
            step(2 * i, 0)
            step(2 * i + 1, 1)
            return 0
        lax.fori_loop(0, nfull // 2, loop_body, 0)

        if nfull % 2:
            ci = nfull - 1
            wait_gathers(0)
            snap_didx(0)
            if ci >= 1:
                wait_scatter(1)
            body(ts0, od0, lane, chunk)
            issue_scatter(0)
            wait_scatter(0)
        else:
            wait_scatter(1)

        if tail:
            b = base0 + nfull * chunk
            pltpu.sync_copy(src_hbm.at[pl.ds(b, tail)], tsi)
            pltpu.sync_copy(dst_hbm.at[pl.ds(b, tail)], tdi)
            pltpu.async_copy(tab_hbm.at[tsi], ts0.at[pl.ds(0, tail)],
                             semg0).wait()
            pltpu.async_copy(om_hbm.at[tdi], od0.at[pl.ds(0, tail)],
                             semg0).wait()
            body(ts0, od0, lane, tail)
            pltpu.async_copy(ts0.at[pl.ds(0, tail)], acc.at[tdi], sems0,
                             add=True).wait()

        plsc.subcore_barrier()
        for j in range(RPT // ZR):
            r0 = sid * RPT + j * ZR
            pltpu.sync_copy(acc.at[pl.ds(r0, ZR)], out_hbm.at[cid, pl.ds(r0, ZR)])

    return kern


@functools.cache
def _build_edge_kernels():
    mesh = plsc.VectorSubcoreMesh(core_axis_name="c", subcore_axis_name="s")
    params = pltpu.CompilerParams(use_tc_tiling_on_sc=False)

    def make(row, body, chunk):
        tail = EPW - (EPW // chunk) * chunk
        scratch = [
            pltpu.VMEM((chunk,), jnp.int32),
            pltpu.VMEM((chunk,), jnp.int32),
            pltpu.VMEM((chunk, row), jnp.float32),
            pltpu.VMEM((chunk, 16), jnp.float32),
            pltpu.VMEM((chunk,), jnp.int32),
            pltpu.VMEM((chunk,), jnp.int32),
            pltpu.VMEM((chunk,), jnp.int32),
            pltpu.VMEM((chunk, row), jnp.float32),
            pltpu.VMEM((chunk, 16), jnp.float32),
            pltpu.VMEM((chunk,), jnp.int32),
            pltpu.VMEM((max(tail, 8),), jnp.int32),
            pltpu.VMEM((max(tail, 8),), jnp.int32),
            pltpu.VMEM_SHARED((NP, row), jnp.float32),
        ]
        scratch += [pltpu.SemaphoreType.DMA] * 6
        return functools.partial(
            pl.kernel,
            compiler_params=params,
            out_type=jax.ShapeDtypeStruct((2, NP, row), jnp.float32),
            mesh=mesh,
            scratch_types=scratch,
        )(_make_edge_kernel(row, body, chunk))

    edge1 = make(R1, _edge_body_l1, 96)
    edge2 = make(R2, _edge_body_l2, 128)
    return edge1, edge2


def kernel(x, edge_index, W1, att_src1, att_dst1, b1, W2, att_src2, att_dst2, b2):
    f32 = jnp.float32
    W1r = W1.reshape(F_IN, H1, C1)
    S1 = jnp.einsum("fkc,kc->fk", W1r, att_src1)
    D1 = jnp.einsum("fkc,kc->fk", W1r, att_dst1)
    W1aug = jnp.concatenate([W1, S1, D1], axis=1)
    W1d = jnp.concatenate([D1, jnp.zeros((F_IN, 8), f32)], axis=1)

    vs2 = W2 @ att_src2[0]
    vd2 = W2 @ att_dst2[0]
    W2aug = jnp.concatenate(
        [W2, vs2[:, None], jnp.zeros((HID, R2 - NC2 - 1), f32)], axis=1)
    W2d = jnp.concatenate(
        [jnp.zeros((HID, 8), f32), vd2[:, None], jnp.zeros((HID, 7), f32)],
        axis=1)

    E8 = jnp.repeat(jnp.eye(H1, dtype=f32), C1, axis=1)

    bn = 2000
    grid = N // bn

    t1, om1 = pl.pallas_call(
        _mm1_body,
        grid=(grid,),
        in_specs=[
            pl.BlockSpec((bn, F_IN), lambda i: (i, 0)),
            pl.BlockSpec((F_IN, R1), lambda i: (0, 0)),
            pl.BlockSpec((F_IN, 16), lambda i: (0, 0)),
        ],
        out_specs=[
            pl.BlockSpec((bn, R1), lambda i: (i, 0)),
            pl.BlockSpec((bn, 16), lambda i: (i, 0)),
        ],
        out_shape=[
            jax.ShapeDtypeStruct((N, R1), f32),
            jax.ShapeDtypeStruct((N, 16), f32),
        ],
    )(x, W1aug, W1d)

    edge1, edge2 = _build_edge_kernels()
    src, dst = edge_index[0], edge_index[1]
    parts1 = edge1(t1, om1, src, dst)

    t2, om2 = pl.pallas_call(
        _merge1_body,
        grid=(grid,),
        in_specs=[
            pl.BlockSpec((2, bn, R1), lambda i: (0, i, 0)),
            pl.BlockSpec((H1, HID), lambda i: (0, 0)),
            pl.BlockSpec((HID, R2), lambda i: (0, 0)),
            pl.BlockSpec((HID, 16), lambda i: (0, 0)),
            pl.BlockSpec((1, HID), lambda i: (0, 0)),
        ],
        out_specs=[
            pl.BlockSpec((bn, R2), lambda i: (i, 0)),
            pl.BlockSpec((bn, 16), lambda i: (i, 0)),
        ],
        out_shape=[
            jax.ShapeDtypeStruct((N, R2), f32),
            jax.ShapeDtypeStruct((N, 16), f32),
        ],
    )(parts1, E8, W2aug, W2d, b1[None, :])

    parts2 = edge2(t2, om2, src, dst)

    out = pl.pallas_call(
        _final_body,
        grid=(grid,),
        in_specs=[
            pl.BlockSpec((2, bn, R2), lambda i: (0, i, 0)),
            pl.BlockSpec((1, NC2), lambda i: (0, 0)),
        ],
        out_specs=pl.BlockSpec((bn, NC2), lambda i: (i, 0)),
        out_shape=jax.ShapeDtypeStruct((N, NC2), f32),
    )(parts2, b2[None, :])

    return out
